# Initial kernel scaffold; baseline (speedup 1.0000x reference)
import jax
import jax.numpy as jnp
from jax.experimental import pallas as pl


def kernel(nf, ef, edge_index, W_e, b_e, W_a, b_a, W_n, b_n):
    raise NotImplementedError("write your pallas kernel here")


# speedup vs baseline: 1.5959x; 1.5959x over previous
import functools

import jax
import jax.numpy as jnp
from jax import lax
from jax.experimental import pallas as pl
from jax.experimental.pallas import tpu as pltpu
from jax.experimental.pallas import tpu_sc as plsc

N = 10000
E = 320000
D = 128
DP = 144

_NW = 32
_EPW = E // _NW
_C = 80
_NCHUNK = _EPW // _C
_NPAD = 10240
_RPT = _NPAD // 16
_ZR = _C


def _prep_node_body(nf_ref, wes_ref, wed_ref, ps_ref, pd_ref):
    nf = nf_ref[...]
    ps_ref[...] = jnp.dot(nf, wes_ref[...], preferred_element_type=jnp.float32)
    pd_ref[...] = jnp.dot(nf, wed_ref[...], preferred_element_type=jnp.float32)


def _prep_node(nf, wes, wed):
    BN = 2000
    return pl.pallas_call(
        _prep_node_body,
        grid=(N // BN,),
        in_specs=[
            pl.BlockSpec((BN, 128), lambda i: (i, 0)),
            pl.BlockSpec((128, DP), lambda i: (0, 0)),
            pl.BlockSpec((128, DP), lambda i: (0, 0)),
        ],
        out_specs=[
            pl.BlockSpec((BN, DP), lambda i: (i, 0)),
            pl.BlockSpec((BN, DP), lambda i: (i, 0)),
        ],
        out_shape=[
            jax.ShapeDtypeStruct((N, DP), jnp.float32),
            jax.ShapeDtypeStruct((N, DP), jnp.float32),
        ],
    )(nf, wes, wed)


def _prep_edge_body(ef_ref, wee_ref, be_ref, efp_ref):
    efp_ref[...] = (
        jnp.dot(ef_ref[...], wee_ref[...], preferred_element_type=jnp.float32)
        + be_ref[...]
    )


def _prep_edge(ef, wee, be):
    BE = 8000
    return pl.pallas_call(
        _prep_edge_body,
        grid=(E // BE,),
        in_specs=[
            pl.BlockSpec((BE, 16), lambda i: (i, 0)),
            pl.BlockSpec((16, DP), lambda i: (0, 0)),
            pl.BlockSpec((1, DP), lambda i: (0, 0)),
        ],
        out_specs=pl.BlockSpec((BE, DP), lambda i: (i, 0)),
        out_shape=jax.ShapeDtypeStruct((E, DP), jnp.float32),
    )(ef, wee, be)


def _sc_body(
    ps_hbm, pd_hbm, efp_hbm, src_hbm, dst_hbm,
    uef_hbm, p0_hbm, p1_hbm,
    sidx, didx, psv, pdv, efpv, outv, agg,
    sem1, sem2,
):
    c = lax.axis_index("c")
    s = lax.axis_index("s")
    wid = c * 16 + s
    base_e = wid * _EPW
    nbase = s * _RPT

    def _zrow(r, carry):
        for j in range(8):
            outv[r, pl.ds(j * 16, 16)] = jnp.zeros((16,), jnp.float32)
        return carry

    lax.fori_loop(0, _ZR, _zrow, 0)
    for k in range(_RPT // _ZR):
        pltpu.sync_copy(outv, agg.at[pl.ds(nbase + _ZR * k, _ZR)])
    plsc.subcore_barrier()

    def _chunk(i, carry):
        off = base_e + i * _C
        pltpu.sync_copy(src_hbm.at[pl.ds(off, _C)], sidx)
        pltpu.sync_copy(dst_hbm.at[pl.ds(off, _C)], didx)
        pltpu.sync_copy(efp_hbm.at[pl.ds(off, _C)], efpv)
        cp1 = pltpu.async_copy(ps_hbm.at[sidx], psv, sem1)
        cp2 = pltpu.async_copy(pd_hbm.at[didx], pdv, sem2)
        cp1.wait()
        cp2.wait()

        def _edge(e, carry2):
            dsa = pl.ds(128, 16)
            x = psv[e, dsa] + pdv[e, dsa] + efpv[e, dsa]
            av = 1.0 / (1.0 + jnp.exp(-x))
            for j in range(8):
                dsj = pl.ds(j * 16, 16)
                outv[e, dsj] = av * (psv[e, dsj] + pdv[e, dsj] + efpv[e, dsj])
            return carry2

        lax.fori_loop(0, _C, _edge, 0)
        pltpu.sync_copy(outv, uef_hbm.at[pl.ds(off, _C)])
        pltpu.sync_copy(outv, agg.at[didx], add=True)
        return carry

    lax.fori_loop(0, _NCHUNK, _chunk, 0)

    plsc.subcore_barrier()
    for k in range(_RPT // _ZR):
        pltpu.sync_copy(agg.at[pl.ds(nbase + _ZR * k, _ZR)], outv)

        @pl.when(c == 0)
        def _dump0():
            pltpu.sync_copy(outv, p0_hbm.at[pl.ds(nbase + _ZR * k, _ZR)])

        @pl.when(c == 1)
        def _dump1():
            pltpu.sync_copy(outv, p1_hbm.at[pl.ds(nbase + _ZR * k, _ZR)])


_sc_edges = functools.partial(
    pl.kernel,
    mesh=plsc.VectorSubcoreMesh(core_axis_name="c", subcore_axis_name="s"),
    compiler_params=pltpu.CompilerParams(use_tc_tiling_on_sc=False),
    out_type=[
        jax.ShapeDtypeStruct((E, 128), jnp.float32),
        jax.ShapeDtypeStruct((_NPAD, 128), jnp.float32),
        jax.ShapeDtypeStruct((_NPAD, 128), jnp.float32),
    ],
    scratch_types=[
        pltpu.VMEM((_C,), jnp.int32),
        pltpu.VMEM((_C,), jnp.int32),
        pltpu.VMEM((_C, DP), jnp.float32),
        pltpu.VMEM((_C, DP), jnp.float32),
        pltpu.VMEM((_C, DP), jnp.float32),
        pltpu.VMEM((_C, 128), jnp.float32),
        pltpu.VMEM_SHARED((_NPAD, 128), jnp.float32),
        pltpu.SemaphoreType.DMA,
        pltpu.SemaphoreType.DMA,
    ],
)(_sc_body)


def _node_out_body(p0_ref, p1_ref, nf_ref, wn1_ref, wn2_ref, bn_ref, out_ref):
    agg = p0_ref[...] + p1_ref[...]
    out_ref[...] = (
        jnp.dot(agg, wn1_ref[...], preferred_element_type=jnp.float32)
        + jnp.dot(nf_ref[...], wn2_ref[...], preferred_element_type=jnp.float32)
        + bn_ref[...]
    )


def _node_out(p0, p1, nf, wn1, wn2, bn):
    BN = 2000
    return pl.pallas_call(
        _node_out_body,
        grid=(N // BN,),
        in_specs=[
            pl.BlockSpec((BN, 128), lambda i: (i, 0)),
            pl.BlockSpec((BN, 128), lambda i: (i, 0)),
            pl.BlockSpec((BN, 128), lambda i: (i, 0)),
            pl.BlockSpec((128, 128), lambda i: (0, 0)),
            pl.BlockSpec((128, 128), lambda i: (0, 0)),
            pl.BlockSpec((1, 128), lambda i: (0, 0)),
        ],
        out_specs=pl.BlockSpec((BN, 128), lambda i: (i, 0)),
        out_shape=jax.ShapeDtypeStruct((N, 128), jnp.float32),
    )(p0, p1, nf, wn1, wn2, bn)


def kernel(nf, ef, edge_index, W_e, b_e, W_a, b_a, W_n, b_n):
    src = edge_index[0].astype(jnp.int32)
    dst = edge_index[1].astype(jnp.int32)
    wes = jnp.concatenate([W_e[:128]] + [W_a[:128]] * 16, axis=1)
    wed = jnp.concatenate([W_e[128:256]] + [W_a[128:256]] * 16, axis=1)
    wee = jnp.concatenate([W_e[256:]] + [W_a[256:]] * 16, axis=1)
    be = jnp.concatenate([b_e, jnp.tile(b_a, 16)]).reshape(1, DP)

    ps, pd = _prep_node(nf, wes, wed)
    efp = _prep_edge(ef, wee, be)

    uef, p0, p1 = _sc_edges(ps, pd, efp, src, dst)
    unf = _node_out(p0, p1, nf, W_n[:128], W_n[128:], b_n.reshape(1, 128))
    return unf, uef
